# baseline (speedup 1.0000x reference)
import functools

import jax
import jax.numpy as jnp
from jax import lax
from jax.experimental import pallas as pl
from jax.experimental.pallas import tpu as pltpu
from jax.experimental.pallas import tpu_sc as plsc

N_EMBD = 64
HALF = N_EMBD // 2
LANES = 16
NC = 2
NS = 16
NW = NC * NS

CH = 512
SUB = 128
NSUB = CH // SUB
UNROLL = 4


def _fire_gather(tlo, thi, idx_all, lo, hi, sem, g):
    for j in range(NSUB):
        isl = idx_all.at[pl.ds(g * CH + j * SUB, SUB)]
        pltpu.async_copy(tlo.at[isl], lo.at[pl.ds(j * SUB, SUB)], sem)
        pltpu.async_copy(thi.at[isl], hi.at[pl.ds(j * SUB, SUB)], sem)


def _wait_gather(tlo, thi, idx_all, lo, hi, sem):
    for j in range(NSUB):
        isl = idx_all.at[pl.ds(j * SUB, SUB)]
        pltpu.make_async_copy(tlo.at[isl], lo.at[pl.ds(j * SUB, SUB)], sem).wait()
        pltpu.make_async_copy(thi.at[isl], hi.at[pl.ds(j * SUB, SUB)], sem).wait()


def _fire_out(lo, hi, out_lo, out_hi, sem, base):
    pltpu.async_copy(lo, out_lo.at[pl.ds(base, CH)], sem)
    pltpu.async_copy(hi, out_hi.at[pl.ds(base, CH)], sem)


def _wait_out(lo, hi, out_lo, out_hi, sem):
    pltpu.make_async_copy(lo, out_lo.at[pl.ds(0, CH)], sem).wait()
    pltpu.make_async_copy(hi, out_hi.at[pl.ds(0, CH)], sem).wait()


def _compute(lo, hi):
    ii = lax.iota(jnp.int32, LANES)

    def quad(r, rcarry):
        rb = r * UNROLL
        for k in range(UNROLL):
            row = rb + k
            va = lo[row, pl.ds(0, LANES)]
            vb = lo[row, pl.ds(LANES, LANES)]
            vc = hi[row, pl.ds(0, LANES)]
            vd = hi[row, pl.ds(LANES, LANES)]
            s = va * va + vb * vb + vc * vc + vd * vd
            for step in (8, 4, 2, 1):
                s = s + s.at[ii ^ step].get(mode="promise_in_bounds")
            i = lax.bitcast_convert_type(s, jnp.int32)
            i = jnp.full((LANES,), 0x5F3759DF, jnp.int32) - lax.shift_right_logical(i, 1)
            y = lax.bitcast_convert_type(i, jnp.float32)
            h = 0.5 * s
            y = y * (1.5 - h * y * y)
            y = y * (1.5 - h * y * y)
            y = y * (1.5 - h * y * y)
            lo[row, pl.ds(0, LANES)] = va * y
            lo[row, pl.ds(LANES, LANES)] = vb * y
            hi[row, pl.ds(0, LANES)] = vc * y
            hi[row, pl.ds(LANES, LANES)] = vd * y
        return rcarry

    lax.fori_loop(0, CH // UNROLL, quad, 0)


def _body(x_hbm, tlo_hbm, thi_hbm, out_lo, out_hi, idx_all,
          lo0, lo1, hi0, hi1, gsem0, gsem1, osem0, osem1):
    wid = lax.axis_index("s") * NC + lax.axis_index("c")
    b_per_w = x_hbm.shape[0] // NW
    nch = b_per_w // CH
    base = wid * b_per_w

    lo = (lo0, lo1)
    hi = (hi0, hi1)
    gsem = (gsem0, gsem1)
    osem = (osem0, osem1)

    pltpu.sync_copy(x_hbm.at[pl.ds(base, b_per_w)], idx_all)

    _fire_gather(tlo_hbm, thi_hbm, idx_all, lo0, hi0, gsem0, 0)
    _fire_gather(tlo_hbm, thi_hbm, idx_all, lo1, hi1, gsem1, 1)
    _wait_gather(tlo_hbm, thi_hbm, idx_all, lo0, hi0, gsem0)
    _compute(lo0, hi0)
    _fire_out(lo0, hi0, out_lo, out_hi, osem0, base)

    def pair(i, carry):
        for off in range(2):
            g = 1 + 2 * i + off
            b = (1 + off) % 2
            nb = 1 - b
            _wait_out(lo[nb], hi[nb], out_lo, out_hi, osem[nb])
            _fire_gather(tlo_hbm, thi_hbm, idx_all, lo[nb], hi[nb], gsem[nb], g + 1)
            _wait_gather(tlo_hbm, thi_hbm, idx_all, lo[b], hi[b], gsem[b])
            _compute(lo[b], hi[b])
            _fire_out(lo[b], hi[b], out_lo, out_hi, osem[b], base + g * CH)
        return carry

    lax.fori_loop(0, (nch - 2) // 2, pair, 0)

    gl = nch - 1
    bl = gl % 2
    _wait_gather(tlo_hbm, thi_hbm, idx_all, lo[bl], hi[bl], gsem[bl])
    _compute(lo[bl], hi[bl])
    _fire_out(lo[bl], hi[bl], out_lo, out_hi, osem[bl], base + gl * CH)
    _wait_out(lo[0], hi[0], out_lo, out_hi, osem[0])
    _wait_out(lo[1], hi[1], out_lo, out_hi, osem[1])


def kernel(x, table):
    B = x.shape[0] * x.shape[1]
    b_per_w = B // NW
    nch = b_per_w // CH
    assert B % NW == 0 and b_per_w % CH == 0 and nch % 2 == 0 and nch >= 4
    xf = jnp.reshape(x, (B,)).astype(jnp.int32)
    tlo = table[:, :HALF]
    thi = table[:, HALF:]
    mesh = plsc.VectorSubcoreMesh(core_axis_name="c", subcore_axis_name="s")
    run = functools.partial(
        pl.kernel,
        out_type=(
            jax.ShapeDtypeStruct((B, HALF), jnp.float32),
            jax.ShapeDtypeStruct((B, HALF), jnp.float32),
        ),
        mesh=mesh,
        scratch_types=[
            pltpu.VMEM((b_per_w,), jnp.int32),
            pltpu.VMEM((CH, HALF), jnp.float32),
            pltpu.VMEM((CH, HALF), jnp.float32),
            pltpu.VMEM((CH, HALF), jnp.float32),
            pltpu.VMEM((CH, HALF), jnp.float32),
            pltpu.SemaphoreType.DMA,
            pltpu.SemaphoreType.DMA,
            pltpu.SemaphoreType.DMA,
            pltpu.SemaphoreType.DMA,
        ],
        compiler_params=pltpu.CompilerParams(use_tc_tiling_on_sc=False),
    )(_body)
    out_lo, out_hi = run(xf, tlo, thi)
    o3_lo = jnp.reshape(out_lo, (x.shape[0], x.shape[1], HALF))
    o3_hi = jnp.reshape(out_hi, (x.shape[0], x.shape[1], HALF))
    return jnp.concatenate([o3_lo, o3_hi], axis=2)


# speedup vs baseline: 1.8243x; 1.8243x over previous
import functools

import jax
import jax.numpy as jnp
from jax import lax
from jax.experimental import pallas as pl
from jax.experimental.pallas import tpu as pltpu
from jax.experimental.pallas import tpu_sc as plsc

N_EMBD = 64
LANES = 16
NC = 2
NS = 16
NW = NC * NS

CH = 512
SUB = 128
NSUB = CH // SUB
UNROLL = 4


def _fire_gather(table_hbm, idx_all, rows, sem, g):
    for j in range(NSUB):
        pltpu.async_copy(
            table_hbm.at[idx_all.at[pl.ds(g * CH + j * SUB, SUB)]],
            rows.at[pl.ds(j * SUB, SUB)],
            sem,
        )


def _wait_gather(table_hbm, idx_all, rows, sem):
    for j in range(NSUB):
        pltpu.make_async_copy(
            table_hbm.at[idx_all.at[pl.ds(j * SUB, SUB)]],
            rows.at[pl.ds(j * SUB, SUB)],
            sem,
        ).wait()


def _wait_out(rows, out_hbm, sem):
    pltpu.make_async_copy(rows, out_hbm.at[pl.ds(0, CH)], sem).wait()


def _compute(rows):
    ii = lax.iota(jnp.int32, LANES)

    def quad(r, rcarry):
        rb = r * UNROLL
        for k in range(UNROLL):
            row = rb + k
            va = rows[row, pl.ds(0, LANES)]
            vb = rows[row, pl.ds(LANES, LANES)]
            vc = rows[row, pl.ds(2 * LANES, LANES)]
            vd = rows[row, pl.ds(3 * LANES, LANES)]
            s = va * va + vb * vb + vc * vc + vd * vd
            for step in (8, 4, 2, 1):
                s = s + s.at[ii ^ step].get(mode="promise_in_bounds")
            i = lax.bitcast_convert_type(s, jnp.int32)
            i = jnp.full((LANES,), 0x5F3759DF, jnp.int32) - lax.shift_right_logical(i, 1)
            y = lax.bitcast_convert_type(i, jnp.float32)
            h = 0.5 * s
            y = y * (1.5 - h * y * y)
            y = y * (1.5 - h * y * y)
            y = y * (1.5 - h * y * y)
            rows[row, pl.ds(0, LANES)] = va * y
            rows[row, pl.ds(LANES, LANES)] = vb * y
            rows[row, pl.ds(2 * LANES, LANES)] = vc * y
            rows[row, pl.ds(3 * LANES, LANES)] = vd * y
        return rcarry

    lax.fori_loop(0, CH // UNROLL, quad, 0)


def _body(x_hbm, table_hbm, out_hbm, idx_all, rows0, rows1,
          gsem0, gsem1, osem0, osem1):
    wid = lax.axis_index("s") * NC + lax.axis_index("c")
    b_per_w = x_hbm.shape[0] // NW
    nch = b_per_w // CH
    base = wid * b_per_w

    rows = (rows0, rows1)
    gsem = (gsem0, gsem1)
    osem = (osem0, osem1)

    pltpu.sync_copy(x_hbm.at[pl.ds(base, b_per_w)], idx_all)

    _fire_gather(table_hbm, idx_all, rows0, gsem0, 0)
    _fire_gather(table_hbm, idx_all, rows1, gsem1, 1)
    _wait_gather(table_hbm, idx_all, rows0, gsem0)
    _compute(rows0)
    pltpu.async_copy(rows0, out_hbm.at[pl.ds(base, CH)], osem0)

    def pair(i, carry):
        for off in range(2):
            g = 1 + 2 * i + off
            b = (1 + off) % 2
            nb = 1 - b
            _wait_out(rows[nb], out_hbm, osem[nb])
            _fire_gather(table_hbm, idx_all, rows[nb], gsem[nb], g + 1)
            _wait_gather(table_hbm, idx_all, rows[b], gsem[b])
            _compute(rows[b])
            pltpu.async_copy(rows[b], out_hbm.at[pl.ds(base + g * CH, CH)], osem[b])
        return carry

    lax.fori_loop(0, (nch - 2) // 2, pair, 0)

    gl = nch - 1
    bl = gl % 2
    _wait_gather(table_hbm, idx_all, rows[bl], gsem[bl])
    _compute(rows[bl])
    pltpu.async_copy(rows[bl], out_hbm.at[pl.ds(base + gl * CH, CH)], osem[bl])
    _wait_out(rows[0], out_hbm, osem[0])
    _wait_out(rows[1], out_hbm, osem[1])


def kernel(x, table):
    B = x.shape[0] * x.shape[1]
    b_per_w = B // NW
    nch = b_per_w // CH
    assert B % NW == 0 and b_per_w % CH == 0 and nch % 2 == 0 and nch >= 4
    xf = jnp.reshape(x, (B,)).astype(jnp.int32)
    mesh = plsc.VectorSubcoreMesh(core_axis_name="c", subcore_axis_name="s")
    run = functools.partial(
        pl.kernel,
        out_type=jax.ShapeDtypeStruct((B, N_EMBD), jnp.float32),
        mesh=mesh,
        scratch_types=[
            pltpu.VMEM((b_per_w,), jnp.int32),
            pltpu.VMEM((CH, N_EMBD), jnp.float32),
            pltpu.VMEM((CH, N_EMBD), jnp.float32),
            pltpu.SemaphoreType.DMA,
            pltpu.SemaphoreType.DMA,
            pltpu.SemaphoreType.DMA,
            pltpu.SemaphoreType.DMA,
        ],
        compiler_params=pltpu.CompilerParams(use_tc_tiling_on_sc=False),
    )(_body)
    out = run(xf, table)
    return jnp.reshape(out, (x.shape[0], x.shape[1], N_EMBD))
